# baseline (speedup 1.0000x reference)
import functools

import jax
import jax.numpy as jnp
from jax import lax
from jax.experimental import pallas as pl
from jax.experimental.pallas import tpu as pltpu
from jax.experimental.pallas import tpu_sc as plsc

_K_BY_HEAD = (10, 20, 40, 500)
_N = 2048
_NCHUNK = _N // 16
_NW = 32
_ROWS_PER_W = 8 * 4 * _N // _NW


def _mono_key(x):
    b = lax.bitcast_convert_type(x, jnp.int32)
    return jnp.where(b >= 0, b, b ^ jnp.int32(0x7FFFFFFF))


def _sc_body(att_hbm, out_hbm, in_buf, out_buf, key_buf, hist, sem):
    del sem
    total_rows, n = att_hbm.shape
    nchunk = n // 16
    rows_per_w = total_rows // _NW
    wid = lax.axis_index("c") * 16 + lax.axis_index("s")
    row0 = wid * rows_per_w
    head = lax.rem(row0 // n, 4)
    k0 = jnp.where(
        head == 0, _K_BY_HEAD[0],
        jnp.where(head == 1, _K_BY_HEAD[1],
                  jnp.where(head == 2, _K_BY_HEAD[2], _K_BY_HEAD[3])))
    k0 = jnp.minimum(k0, n).astype(jnp.int32)

    def scan_hist(rem_k):
        def sweep(j, carry):
            cnt_true, carry_above = carry
            v = 15 - j
            h = hist[pl.ds(v * 16, 16)]
            p_incl = plsc.cumsum(h)
            tot = jnp.sum(h)
            suffix_incl = carry_above + (tot - p_incl) + h
            mask = suffix_incl >= rem_k
            cnt_true = cnt_true + jnp.max(plsc.all_reduce_population_count(mask))
            return cnt_true, carry_above + tot

        cnt_true, _ = lax.fori_loop(0, 16, sweep, (jnp.int32(0), jnp.int32(0)))
        bstar = cnt_true - 1

        def gt_sum(v, acc):
            h = hist[pl.ds(v * 16, 16)]
            idx = jax.lax.iota(jnp.int32, 16) + v * 16
            return acc + jnp.sum(jnp.where(idx > bstar, h, 0))

        cnt_gt = lax.fori_loop(0, 16, gt_sum, jnp.int32(0))
        return bstar, cnt_gt

    def zero_hist():
        def z(v, _):
            hist[pl.ds(v * 16, 16)] = jnp.zeros((16,), jnp.int32)
            return 0
        lax.fori_loop(0, 16, z, 0)

    ones16 = jnp.ones((16,), jnp.int32)

    def per_row(i, _):
        row = row0 + i
        pltpu.sync_copy(att_hbm.at[pl.ds(row, 1)], in_buf)

        def maxbody(c, acc):
            x = in_buf[0, pl.ds(c * 16, 16)]
            key_buf[pl.ds(c * 16, 16)] = _mono_key(x)
            return jnp.maximum(acc, x)

        acc = lax.fori_loop(0, nchunk, maxbody,
                            jnp.full((16,), -3.4e38, jnp.float32), unroll=4)
        m = jnp.max(acc)

        zero_hist()

        def pass0(c, _):
            key = key_buf[pl.ds(c * 16, 16)]
            bkt = (key >> 24) + 128
            plsc.addupdate_scatter(hist, [bkt], ones16)
            return 0

        lax.fori_loop(0, nchunk, pass0, 0, unroll=4)
        b0, gt0 = scan_hist(k0)
        prefix = b0 - 128
        rem_k = k0 - gt0

        def radix_pass(shift, prefix, rem_k):
            zero_hist()

            def body(c, _):
                key = key_buf[pl.ds(c * 16, 16)]
                match = (key >> (shift + 8)) == prefix
                bkt = (key >> shift) & 0xFF
                plsc.addupdate_scatter(hist, [bkt], ones16, mask=match)
                return 0

            lax.fori_loop(0, nchunk, body, 0, unroll=4)
            b, gt = scan_hist(rem_k)
            return (prefix << 8) | b, rem_k - gt

        prefix, rem_k = radix_pass(16, prefix, rem_k)
        prefix, rem_k = radix_pass(8, prefix, rem_k)
        t, _ = radix_pass(0, prefix, rem_k)

        def passA(c, z):
            x = in_buf[0, pl.ds(c * 16, 16)]
            key = key_buf[pl.ds(c * 16, 16)]
            e = jnp.exp(x - m)
            em = jnp.where(key >= t, e, 0.0)
            out_buf[0, pl.ds(c * 16, 16)] = em
            return z + em

        z16 = lax.fori_loop(0, nchunk, passA, jnp.zeros((16,), jnp.float32),
                            unroll=4)
        invz = jnp.ones((16,), jnp.float32) / jnp.sum(z16)

        def passB(c, _):
            out_buf[0, pl.ds(c * 16, 16)] = out_buf[0, pl.ds(c * 16, 16)] * invz
            return 0

        lax.fori_loop(0, nchunk, passB, 0, unroll=4)
        pltpu.sync_copy(out_buf, out_hbm.at[pl.ds(row, 1)])
        return 0

    lax.fori_loop(0, rows_per_w, per_row, 0)


def kernel(attention):
    bsz, heads, n, _ = attention.shape
    att2 = attention.reshape(bsz * heads * n, n)
    mesh = plsc.VectorSubcoreMesh(core_axis_name="c", subcore_axis_name="s")
    out2 = pl.kernel(
        _sc_body,
        out_type=jax.ShapeDtypeStruct(att2.shape, att2.dtype),
        mesh=mesh,
        compiler_params=pltpu.CompilerParams(needs_layout_passes=False),
        scratch_types=[
            pltpu.VMEM((1, n), jnp.float32),
            pltpu.VMEM((1, n), jnp.float32),
            pltpu.VMEM((n,), jnp.int32),
            pltpu.VMEM((256,), jnp.int32),
            pltpu.SemaphoreType.DMA,
        ],
    )(att2)
    return out2.reshape(attention.shape)


# speedup vs baseline: 1.1409x; 1.1409x over previous
import jax
import jax.numpy as jnp
from jax import lax
from jax.experimental import pallas as pl
from jax.experimental.pallas import tpu as pltpu
from jax.experimental.pallas import tpu_sc as plsc

_K_BY_HEAD = (10, 20, 40, 500)
_NW = 32
_BATCH = 8


def _mono_key(x):
    b = lax.bitcast_convert_type(x, jnp.int32)
    return jnp.where(b >= 0, b, b ^ jnp.int32(0x7FFFFFFF))


def _sc_body(att_hbm, out_hbm, in_buf, out_buf, key_buf, hist,
             sem_in0, sem_in1, sem_out0, sem_out1):
    total_rows, n = att_hbm.shape
    nchunk = n // 16
    rows_per_w = total_rows // _NW
    nb = rows_per_w // _BATCH
    wid = lax.axis_index("c") * 16 + lax.axis_index("s")
    row0 = wid * rows_per_w
    head = lax.rem(row0 // n, 4)
    k0 = jnp.where(
        head == 0, _K_BY_HEAD[0],
        jnp.where(head == 1, _K_BY_HEAD[1],
                  jnp.where(head == 2, _K_BY_HEAD[2], _K_BY_HEAD[3])))
    k0 = jnp.minimum(k0, n).astype(jnp.int32)
    sems_in = (sem_in0, sem_in1)
    sems_out = (sem_out0, sem_out1)

    def scan_hist(rem_k):
        def sweep(j, carry):
            cnt_true, carry_above, gt_acc = carry
            v = 15 - j
            h = hist[pl.ds(v * 16, 16)]
            p_incl = plsc.cumsum(h)
            tot = jnp.sum(h)
            suffix_incl = carry_above + (tot - p_incl) + h
            mask = suffix_incl >= rem_k
            cnt_true = cnt_true + jnp.max(plsc.all_reduce_population_count(mask))
            gt_acc = gt_acc + jnp.where(mask, 0, h)
            return cnt_true, carry_above + tot, gt_acc

        cnt_true, _, gt_acc = lax.fori_loop(
            0, 16, sweep,
            (jnp.int32(0), jnp.int32(0), jnp.zeros((16,), jnp.int32)),
            unroll=4)
        return cnt_true - 1, jnp.sum(gt_acc)

    def zero_hist():
        def z(v, _):
            hist[pl.ds(v * 16, 16)] = jnp.zeros((16,), jnp.int32)
            return 0
        lax.fori_loop(0, 16, z, 0, unroll=8)

    ones16 = jnp.ones((16,), jnp.int32)

    def in_dma(b, sl):
        return pltpu.make_async_copy(
            att_hbm.at[pl.ds(row0 + b * _BATCH, _BATCH)], in_buf.at[sl],
            sems_in[sl])

    def out_dma(b, sl):
        return pltpu.make_async_copy(
            out_buf.at[sl], out_hbm.at[pl.ds(row0 + b * _BATCH, _BATCH)],
            sems_out[sl])

    def per_row(sl, r):
        zero_hist()

        def pass0(c, acc):
            x = in_buf[sl, r, pl.ds(c * 16, 16)]
            key = _mono_key(x)
            key_buf[pl.ds(c * 16, 16)] = key
            bkt = (key >> 24) + 128
            plsc.addupdate_scatter(hist, [bkt], ones16)
            return jnp.maximum(acc, x)

        acc = lax.fori_loop(0, nchunk, pass0,
                            jnp.full((16,), -3.4e38, jnp.float32), unroll=4)
        m = jnp.max(acc)
        b0, gt0 = scan_hist(k0)
        prefix = b0 - 128
        rem_k = k0 - gt0

        def radix_pass(shift, prefix, rem_k):
            zero_hist()

            def body(c, _):
                key = key_buf[pl.ds(c * 16, 16)]
                match = (key >> (shift + 8)) == prefix
                bkt = (key >> shift) & 0xFF
                plsc.addupdate_scatter(hist, [bkt], ones16, mask=match)
                return 0

            lax.fori_loop(0, nchunk, body, 0, unroll=4)
            b, gt = scan_hist(rem_k)
            return (prefix << 8) | b, rem_k - gt

        prefix, rem_k = radix_pass(16, prefix, rem_k)
        prefix, rem_k = radix_pass(8, prefix, rem_k)
        t, _ = radix_pass(0, prefix, rem_k)

        def passA(c, z):
            x = in_buf[sl, r, pl.ds(c * 16, 16)]
            key = key_buf[pl.ds(c * 16, 16)]
            e = jnp.exp(x - m)
            em = jnp.where(key >= t, e, 0.0)
            out_buf[sl, r, pl.ds(c * 16, 16)] = em
            return z + em

        z16 = lax.fori_loop(0, nchunk, passA, jnp.zeros((16,), jnp.float32),
                            unroll=4)
        invz = jnp.ones((16,), jnp.float32) / jnp.sum(z16)

        def passB(c, _):
            out_buf[sl, r, pl.ds(c * 16, 16)] = (
                out_buf[sl, r, pl.ds(c * 16, 16)] * invz)
            return 0

        lax.fori_loop(0, nchunk, passB, 0, unroll=4)

    in_dma(0, 0).start()
    in_dma(1, 1).start()

    def per_batch_pair(p, _):
        for sl in (0, 1):
            b = 2 * p + sl
            in_dma(b, sl).wait()

            @pl.when(b >= 2)
            def _():
                out_dma(b - 2, sl).wait()

            def row_body(r, _):
                per_row(sl, r)
                return 0

            lax.fori_loop(0, _BATCH, row_body, 0)
            out_dma(b, sl).start()

            @pl.when(b + 2 < nb)
            def _():
                in_dma(b + 2, sl).start()
        return 0

    lax.fori_loop(0, nb // 2, per_batch_pair, 0)
    out_dma(nb - 2, 0).wait()
    out_dma(nb - 1, 1).wait()


def kernel(attention):
    bsz, heads, n, _ = attention.shape
    att2 = attention.reshape(bsz * heads * n, n)
    mesh = plsc.VectorSubcoreMesh(core_axis_name="c", subcore_axis_name="s")
    out2 = pl.kernel(
        _sc_body,
        out_type=jax.ShapeDtypeStruct(att2.shape, att2.dtype),
        mesh=mesh,
        compiler_params=pltpu.CompilerParams(needs_layout_passes=False),
        scratch_types=[
            pltpu.VMEM((2, _BATCH, n), jnp.float32),
            pltpu.VMEM((2, _BATCH, n), jnp.float32),
            pltpu.VMEM((n,), jnp.int32),
            pltpu.VMEM((256,), jnp.int32),
            pltpu.SemaphoreType.DMA,
            pltpu.SemaphoreType.DMA,
            pltpu.SemaphoreType.DMA,
            pltpu.SemaphoreType.DMA,
        ],
    )(att2)
    return out2.reshape(attention.shape)
